# baseline (speedup 1.0000x reference)
import jax
import jax.numpy as jnp
from jax import lax
from jax.experimental import pallas as pl
from jax.experimental.pallas import tpu as pltpu
from jax.experimental.pallas import tpu_sc as plsc

_D = 128
_B = 10
_NPG = 1000
_EPG = 32000
_N = _B * _NPG
_E = _B * _EPG

_TE = 2000
_NT = _E // _TE
_TPG = _EPG // _TE

_NC = 2
_NS = 16
_NW = _NC * _NS
_CHUNK = 400
_PERW = _E // _NW
_NPV = 1024

_EPS = 1e-5
_BF = jnp.bfloat16
_F32 = jnp.float32


def _softplus(x):
    return jnp.maximum(x, 0.0) + jnp.log(1.0 + jnp.exp(-jnp.abs(x)))


def _bdot(a, b):
    return jnp.dot(a.astype(_BF), b.astype(_BF), preferred_element_type=_F32)


def _sc_gather(node_feats, src, dst):
    mesh = plsc.VectorSubcoreMesh(core_axis_name="c", subcore_axis_name="s")

    @pl.kernel(
        out_type=(
            jax.ShapeDtypeStruct((_E, _D), _F32),
            jax.ShapeDtypeStruct((_E, _D), _F32),
        ),
        mesh=mesh,
        scratch_types=[
            pltpu.VMEM((_CHUNK,), jnp.int32),
            pltpu.VMEM((_CHUNK, _D), _F32),
            pltpu.VMEM((_CHUNK,), jnp.int32),
            pltpu.VMEM((_CHUNK, _D), _F32),
        ],
    )
    def k(nf_hbm, src_hbm, dst_hbm, xs_hbm, xd_hbm, idx_s, rows_s, idx_d, rows_d):
        wid = lax.axis_index("s") * _NC + lax.axis_index("c")
        base = wid * _PERW

        @pl.loop(0, _PERW // _CHUNK)
        def _(j):
            off = base + j * _CHUNK
            pltpu.sync_copy(src_hbm.at[pl.ds(off, _CHUNK)], idx_s)
            pltpu.sync_copy(nf_hbm.at[idx_s], rows_s)
            pltpu.sync_copy(rows_s, xs_hbm.at[pl.ds(off, _CHUNK)])
            pltpu.sync_copy(dst_hbm.at[pl.ds(off, _CHUNK)], idx_d)
            pltpu.sync_copy(nf_hbm.at[idx_d], rows_d)
            pltpu.sync_copy(rows_d, xd_hbm.at[pl.ds(off, _CHUNK)])

    return k(node_feats, src, dst)


def _edge_stats_body(xs, xd, ef, g3, ws, wd, we, wg, b1, s1, s2):
    i = pl.program_id(0)
    grow = _bdot(g3[0], wg[...]) + b1[...]
    h = _bdot(xs[...], ws[...]) + _bdot(xd[...], wd[...]) \
        + _bdot(ef[...], we[...]) + grow

    @pl.when(i == 0)
    def _():
        s1[...] = jnp.zeros_like(s1)
        s2[...] = jnp.zeros_like(s2)

    s1[...] += jnp.sum(h, axis=0, keepdims=True)
    s2[...] += jnp.sum(h * h, axis=0, keepdims=True)


def _edge_main_body(xs, xd, ef, g3, ws, wd, we, wg, b1, k1c1, w2, b2,
                    h2o, s1, s2):
    i = pl.program_id(0)
    grow = _bdot(g3[0], wg[...]) + b1[...]
    h = _bdot(xs[...], ws[...]) + _bdot(xd[...], wd[...]) \
        + _bdot(ef[...], we[...]) + grow
    a = _softplus(h * k1c1[0:1, :] + k1c1[1:2, :])
    h2 = _bdot(a, w2[...]) + b2[...]
    h2o[...] = h2.astype(_BF)

    @pl.when(i == 0)
    def _():
        s1[...] = jnp.zeros_like(s1)
        s2[...] = jnp.zeros_like(s2)

    s1[...] += jnp.sum(h2, axis=0, keepdims=True)
    s2[...] += jnp.sum(h2 * h2, axis=0, keepdims=True)


def _edge_final_body(h2, dstl, k2c2, he, psum, pcnt):
    i = pl.program_id(0)
    hev = _softplus(h2[...].astype(_F32) * k2c2[0:1, :] + k2c2[1:2, :])
    he[...] = hev

    col = dstl[0, 0, :].reshape(_TE, 1) - (i // _TPG) * _NPG
    oh = (col == lax.broadcasted_iota(jnp.int32, (_TE, _NPV), 1))

    @pl.when(i % _TPG == 0)
    def _():
        psum[...] = jnp.zeros_like(psum)
        pcnt[...] = jnp.zeros_like(pcnt)

    dn = (((0,), (0,)), ((), ()))
    part = lax.dot_general(oh.astype(_BF), hev.astype(_BF), dn,
                           preferred_element_type=_F32)
    psum[0] += part
    pcnt[0] += jnp.sum(oh.astype(_F32), axis=0, keepdims=True)


def _node_global_body(nf, ssum, cntb, gp,
                      wn1a, wn1b, wn1c, bn1, gn1, ben1,
                      wn2, bn2, gn2, ben2,
                      wg1a, wg1b, wg1c, bg1, gg1, beg1,
                      wg2, bg2, gg2, beg2,
                      hn_o, hg_o):
    cnt = jnp.maximum(cntb[...], 1.0)
    have = ssum[...] / cnt

    r = lax.broadcasted_iota(jnp.int32, (_N, 16), 0)
    c = lax.broadcasted_iota(jnp.int32, (_N, 16), 1)
    oh = (r // _NPG == c).astype(_BF)

    gtab = _bdot(gp[...], wn1c[...])
    h1 = _bdot(nf[...], wn1a[...]) + _bdot(have, wn1b[...]) \
        + _bdot(oh, gtab) + bn1[...]
    mu = jnp.mean(h1, axis=0, keepdims=True)
    var = jnp.mean(h1 * h1, axis=0, keepdims=True) - mu * mu
    a1 = _softplus(gn1[...] * (h1 - mu) * jax.lax.rsqrt(var + _EPS) + ben1[...])

    h2 = _bdot(a1, wn2[...]) + bn2[...]
    mu2 = jnp.mean(h2, axis=0, keepdims=True)
    var2 = jnp.mean(h2 * h2, axis=0, keepdims=True) - mu2 * mu2
    hn = _softplus(gn2[...] * (h2 - mu2) * jax.lax.rsqrt(var2 + _EPS) + ben2[...])
    hn_o[...] = hn

    dn = (((0,), (0,)), ((), ()))
    han = lax.dot_general(oh, hn.astype(_BF), dn,
                          preferred_element_type=_F32) * (1.0 / _NPG)
    hae = lax.dot_general(oh, have.astype(_BF), dn,
                          preferred_element_type=_F32) * (1.0 / _NPG)

    hg1 = _bdot(han, wg1a[...]) + _bdot(hae, wg1b[...]) \
        + _bdot(gp[...], wg1c[...]) + bg1[...]
    m = (lax.broadcasted_iota(jnp.int32, (16, 384), 0) < _B).astype(_F32)
    mug = jnp.sum(hg1 * m, axis=0, keepdims=True) / _B
    varg = jnp.sum((hg1 - mug) * (hg1 - mug) * m, axis=0, keepdims=True) / _B
    ag = _softplus(gg1[...] * (hg1 - mug) * jax.lax.rsqrt(varg + _EPS) + beg1[...])

    hg2 = _bdot(ag, wg2[...]) + bg2[...]
    m2 = (lax.broadcasted_iota(jnp.int32, (16, _D), 0) < _B).astype(_F32)
    mug2 = jnp.sum(hg2 * m2, axis=0, keepdims=True) / _B
    varg2 = jnp.sum((hg2 - mug2) * (hg2 - mug2) * m2, axis=0, keepdims=True) / _B
    hg_o[...] = _softplus(gg2[...] * (hg2 - mug2) * jax.lax.rsqrt(varg2 + _EPS)
                          + beg2[...])


def _row(x):
    return x.reshape(1, -1)


def kernel(node_feats, edge_feats, global_feats, params, src, dst, node_gid,
           batch_num_nodes, batch_num_edges):
    p = params
    w1t = p['e1_W'].T
    ws, wd, we, wg = w1t[0:128], w1t[128:256], w1t[256:384], w1t[384:512]
    b1 = _row(p['e1_b'])
    w2t = p['e2_W'].T
    b2 = _row(p['e2_b'])

    g3 = global_feats.astype(_BF).reshape(_B, 1, _D)

    xs, xd = _sc_gather(node_feats, src, dst)
    efb = edge_feats.astype(_BF)
    ws, wd, we, wg = (w.astype(_BF) for w in (ws, wd, we, wg))
    w2t = w2t.astype(_BF)

    espec = pl.BlockSpec((_TE, _D), lambda i: (i, 0))
    gspec = pl.BlockSpec((1, 1, _D), lambda i: (i // _TPG, 0, 0))
    wspec = pl.BlockSpec((_D, 512), lambda i: (0, 0))
    rspec = lambda n: pl.BlockSpec((1, n), lambda i: (0, 0))

    s1, s2 = pl.pallas_call(
        _edge_stats_body,
        grid=(_NT,),
        in_specs=[espec, espec, espec, gspec, wspec, wspec, wspec, wspec,
                  rspec(512)],
        out_specs=(rspec(512), rspec(512)),
        out_shape=(jax.ShapeDtypeStruct((1, 512), _F32),
                   jax.ShapeDtypeStruct((1, 512), _F32)),
    )(xs, xd, efb, g3, ws, wd, we, wg, b1)

    mu = s1 / _E
    var = s2 / _E - mu * mu
    k1 = p['e1_g'] * jax.lax.rsqrt(var[0] + _EPS)
    c1 = p['e1_be'] - mu[0] * k1
    k1c1 = jnp.concatenate([_row(k1), _row(c1)], axis=0)

    h2, t1, t2 = pl.pallas_call(
        _edge_main_body,
        grid=(_NT,),
        in_specs=[espec, espec, espec, gspec, wspec, wspec, wspec, wspec,
                  rspec(512), pl.BlockSpec((2, 512), lambda i: (0, 0)),
                  pl.BlockSpec((512, _D), lambda i: (0, 0)), rspec(_D)],
        out_specs=(espec, rspec(_D), rspec(_D)),
        out_shape=(jax.ShapeDtypeStruct((_E, _D), _BF),
                   jax.ShapeDtypeStruct((1, _D), _F32),
                   jax.ShapeDtypeStruct((1, _D), _F32)),
    )(xs, xd, efb, g3, ws, wd, we, wg, b1, k1c1, w2t, b2)

    mu2 = t1 / _E
    var2 = t2 / _E - mu2 * mu2
    k2 = p['e2_g'] * jax.lax.rsqrt(var2[0] + _EPS)
    c2 = p['e2_be'] - mu2[0] * k2
    k2c2 = jnp.concatenate([_row(k2), _row(c2)], axis=0)

    dst3 = dst.reshape(_NT, 1, _TE)
    he, gsum, gcnt = pl.pallas_call(
        _edge_final_body,
        grid=(_NT,),
        in_specs=[espec, pl.BlockSpec((1, 1, _TE), lambda i: (i, 0, 0)),
                  pl.BlockSpec((2, _D), lambda i: (0, 0))],
        out_specs=(espec,
                   pl.BlockSpec((1, _NPV, _D), lambda i: (i // _TPG, 0, 0)),
                   pl.BlockSpec((1, 1, _NPV), lambda i: (i // _TPG, 0, 0))),
        out_shape=(jax.ShapeDtypeStruct((_E, _D), _F32),
                   jax.ShapeDtypeStruct((_B, _NPV, _D), _F32),
                   jax.ShapeDtypeStruct((_B, 1, _NPV), _F32)),
    )(h2, dst3, k2c2)

    psums = gsum[:, :_NPG].reshape(_N, _D)
    cnt = gcnt[:, 0, :_NPG].reshape(_N)
    cntb = jnp.broadcast_to(cnt[:, None], (_N, _D))

    gp = jnp.zeros((16, _D), _F32).at[0:_B].set(global_feats)
    wn1t = p['n1_W'].T
    wg1t = p['g1_W'].T
    full = lambda shape: pl.BlockSpec(shape, lambda: tuple(0 for _ in shape))
    args = (node_feats, psums, cntb, gp,
            wn1t[0:128], wn1t[128:256], wn1t[256:384],
            _row(p['n1_b']), _row(p['n1_g']), _row(p['n1_be']),
            p['n2_W'].T, _row(p['n2_b']), _row(p['n2_g']), _row(p['n2_be']),
            wg1t[0:128], wg1t[128:256], wg1t[256:384],
            _row(p['g1_b']), _row(p['g1_g']), _row(p['g1_be']),
            p['g2_W'].T, _row(p['g2_b']), _row(p['g2_g']), _row(p['g2_be']))
    hn, hgp = pl.pallas_call(
        _node_global_body,
        in_specs=[full(a.shape) for a in args],
        out_specs=(full((_N, _D)), full((16, _D))),
        out_shape=(jax.ShapeDtypeStruct((_N, _D), _F32),
                   jax.ShapeDtypeStruct((16, _D), _F32)),
    )(*args)

    return hn, he, hgp[0:_B]


# speedup vs baseline: 1.1255x; 1.1255x over previous
import jax
import jax.numpy as jnp
from jax import lax
from jax.experimental import pallas as pl
from jax.experimental.pallas import tpu as pltpu
from jax.experimental.pallas import tpu_sc as plsc

_D = 128
_B = 10
_NPG = 1000
_EPG = 32000
_N = _B * _NPG
_E = _B * _EPG

_TE = 2000
_NT = _E // _TE
_TPG = _EPG // _TE

_NC = 2
_NS = 16
_NW = _NC * _NS
_CHUNK = 400
_PERW = _E // _NW
_NPV = 1024

_EPS = 1e-5
_BF = jnp.bfloat16
_F32 = jnp.float32


def _softplus(x):
    return jnp.maximum(x, 0.0) + jnp.log(1.0 + jnp.exp(-jnp.abs(x)))


def _bdot(a, b):
    return jnp.dot(a.astype(_BF), b.astype(_BF), preferred_element_type=_F32)


def _sc_gather(node_feats, src, dst):
    mesh = plsc.VectorSubcoreMesh(core_axis_name="c", subcore_axis_name="s")

    @pl.kernel(
        out_type=jax.ShapeDtypeStruct((_E, 2 * _D), _F32),
        mesh=mesh,
        scratch_types=[
            pltpu.VMEM((_CHUNK,), jnp.int32),
            pltpu.VMEM((_CHUNK, _D), _F32),
            pltpu.VMEM((_CHUNK,), jnp.int32),
            pltpu.VMEM((_CHUNK, _D), _F32),
        ],
    )
    def k(nf_hbm, src_hbm, dst_hbm, xsd_hbm, idx_s, rows_s, idx_d, rows_d):
        wid = lax.axis_index("s") * _NC + lax.axis_index("c")
        base = wid * _PERW

        @pl.loop(0, _PERW // _CHUNK)
        def _(j):
            off = base + j * _CHUNK
            pltpu.sync_copy(src_hbm.at[pl.ds(off, _CHUNK)], idx_s)
            pltpu.sync_copy(nf_hbm.at[idx_s], rows_s)
            pltpu.sync_copy(rows_s, xsd_hbm.at[pl.ds(off, _CHUNK), pl.ds(0, _D)])
            pltpu.sync_copy(dst_hbm.at[pl.ds(off, _CHUNK)], idx_d)
            pltpu.sync_copy(nf_hbm.at[idx_d], rows_d)
            pltpu.sync_copy(rows_d, xsd_hbm.at[pl.ds(off, _CHUNK), pl.ds(_D, _D)])

    return k(node_feats, src, dst)


def _edge_stats_body(xsd, ef, g3, wsd, we, wg, b1, s1, s2):
    i = pl.program_id(0)
    grow = _bdot(g3[0], wg[...]) + b1[...]
    h = _bdot(xsd[...], wsd[...]) + (_bdot(ef[...], we[...]) + grow)

    @pl.when(i == 0)
    def _():
        s1[...] = jnp.zeros_like(s1)
        s2[...] = jnp.zeros_like(s2)

    s1[...] += jnp.sum(h, axis=0, keepdims=True)
    s2[...] += jnp.sum(h * h, axis=0, keepdims=True)


def _edge_main_body(xsd, ef, g3, wsd, we, wg, b1, k1c1, w2, b2,
                    h2o, s1, s2):
    i = pl.program_id(0)
    grow = _bdot(g3[0], wg[...]) + b1[...]
    h = _bdot(xsd[...], wsd[...]) + (_bdot(ef[...], we[...]) + grow)
    a = _softplus(h * k1c1[0:1, :] + k1c1[1:2, :])
    h2 = _bdot(a, w2[...]) + b2[...]
    h2o[...] = h2

    @pl.when(i == 0)
    def _():
        s1[...] = jnp.zeros_like(s1)
        s2[...] = jnp.zeros_like(s2)

    s1[...] += jnp.sum(h2, axis=0, keepdims=True)
    s2[...] += jnp.sum(h2 * h2, axis=0, keepdims=True)


def _edge_final_body(h2, dstl, k2c2, he, psum, pcnt):
    i = pl.program_id(0)
    hev = _softplus(h2[...] * k2c2[0:1, :] + k2c2[1:2, :])
    he[...] = hev

    col = dstl[0, 0, :].reshape(_TE, 1) - (i // _TPG) * _NPG
    oh = (col == lax.broadcasted_iota(jnp.int32, (_TE, _NPV), 1))

    @pl.when(i % _TPG == 0)
    def _():
        psum[...] = jnp.zeros_like(psum)
        pcnt[...] = jnp.zeros_like(pcnt)

    dn = (((0,), (0,)), ((), ()))
    part = lax.dot_general(oh.astype(_BF), hev.astype(_BF), dn,
                           preferred_element_type=_F32)
    psum[0] += part
    pcnt[0] += jnp.sum(oh.astype(_F32), axis=0, keepdims=True)


def _node_global_body(nf, ssum, cntb, gp,
                      wn1a, wn1b, wn1c, bn1, gn1, ben1,
                      wn2, bn2, gn2, ben2,
                      wg1a, wg1b, wg1c, bg1, gg1, beg1,
                      wg2, bg2, gg2, beg2,
                      hn_o, hg_o):
    cnt = jnp.maximum(cntb[...], 1.0)
    have = ssum[...] / cnt

    r = lax.broadcasted_iota(jnp.int32, (_N, 16), 0)
    c = lax.broadcasted_iota(jnp.int32, (_N, 16), 1)
    oh = (r // _NPG == c).astype(_BF)

    gtab = _bdot(gp[...], wn1c[...])
    h1 = _bdot(nf[...], wn1a[...]) + _bdot(have, wn1b[...]) \
        + _bdot(oh, gtab) + bn1[...]
    mu = jnp.mean(h1, axis=0, keepdims=True)
    var = jnp.mean(h1 * h1, axis=0, keepdims=True) - mu * mu
    a1 = _softplus(gn1[...] * (h1 - mu) * jax.lax.rsqrt(var + _EPS) + ben1[...])

    h2 = _bdot(a1, wn2[...]) + bn2[...]
    mu2 = jnp.mean(h2, axis=0, keepdims=True)
    var2 = jnp.mean(h2 * h2, axis=0, keepdims=True) - mu2 * mu2
    hn = _softplus(gn2[...] * (h2 - mu2) * jax.lax.rsqrt(var2 + _EPS) + ben2[...])
    hn_o[...] = hn

    dn = (((0,), (0,)), ((), ()))
    han = lax.dot_general(oh, hn.astype(_BF), dn,
                          preferred_element_type=_F32) * (1.0 / _NPG)
    hae = lax.dot_general(oh, have.astype(_BF), dn,
                          preferred_element_type=_F32) * (1.0 / _NPG)

    hg1 = _bdot(han, wg1a[...]) + _bdot(hae, wg1b[...]) \
        + _bdot(gp[...], wg1c[...]) + bg1[...]
    m = (lax.broadcasted_iota(jnp.int32, (16, 384), 0) < _B).astype(_F32)
    mug = jnp.sum(hg1 * m, axis=0, keepdims=True) / _B
    varg = jnp.sum((hg1 - mug) * (hg1 - mug) * m, axis=0, keepdims=True) / _B
    ag = _softplus(gg1[...] * (hg1 - mug) * jax.lax.rsqrt(varg + _EPS) + beg1[...])

    hg2 = _bdot(ag, wg2[...]) + bg2[...]
    m2 = (lax.broadcasted_iota(jnp.int32, (16, _D), 0) < _B).astype(_F32)
    mug2 = jnp.sum(hg2 * m2, axis=0, keepdims=True) / _B
    varg2 = jnp.sum((hg2 - mug2) * (hg2 - mug2) * m2, axis=0, keepdims=True) / _B
    hg_o[...] = _softplus(gg2[...] * (hg2 - mug2) * jax.lax.rsqrt(varg2 + _EPS)
                          + beg2[...])


def _row(x):
    return x.reshape(1, -1)


def kernel(node_feats, edge_feats, global_feats, params, src, dst, node_gid,
           batch_num_nodes, batch_num_edges):
    p = params
    w1t = p['e1_W'].T
    ws, wd, we, wg = w1t[0:128], w1t[128:256], w1t[256:384], w1t[384:512]
    b1 = _row(p['e1_b'])
    w2t = p['e2_W'].T
    b2 = _row(p['e2_b'])

    g3 = global_feats.reshape(_B, 1, _D)

    xsd = _sc_gather(node_feats, src, dst)
    wsd = jnp.concatenate([ws, wd], axis=0)

    espec = pl.BlockSpec((_TE, _D), lambda i: (i, 0))
    espec2 = pl.BlockSpec((_TE, 2 * _D), lambda i: (i, 0))
    gspec = pl.BlockSpec((1, 1, _D), lambda i: (i // _TPG, 0, 0))
    wspec = pl.BlockSpec((_D, 512), lambda i: (0, 0))
    rspec = lambda n: pl.BlockSpec((1, n), lambda i: (0, 0))

    s1, s2 = pl.pallas_call(
        _edge_stats_body,
        grid=(_NT,),
        in_specs=[espec2, espec, gspec, pl.BlockSpec((2 * _D, 512), lambda i: (0, 0)),
                  wspec, wspec, rspec(512)],
        out_specs=(rspec(512), rspec(512)),
        out_shape=(jax.ShapeDtypeStruct((1, 512), _F32),
                   jax.ShapeDtypeStruct((1, 512), _F32)),
    )(xsd, edge_feats, g3, wsd, we, wg, b1)

    mu = s1 / _E
    var = s2 / _E - mu * mu
    k1 = p['e1_g'] * jax.lax.rsqrt(var[0] + _EPS)
    c1 = p['e1_be'] - mu[0] * k1
    k1c1 = jnp.concatenate([_row(k1), _row(c1)], axis=0)

    h2, t1, t2 = pl.pallas_call(
        _edge_main_body,
        grid=(_NT,),
        in_specs=[espec2, espec, gspec, pl.BlockSpec((2 * _D, 512), lambda i: (0, 0)),
                  wspec, wspec, rspec(512), pl.BlockSpec((2, 512), lambda i: (0, 0)),
                  pl.BlockSpec((512, _D), lambda i: (0, 0)), rspec(_D)],
        out_specs=(espec, rspec(_D), rspec(_D)),
        out_shape=(jax.ShapeDtypeStruct((_E, _D), _F32),
                   jax.ShapeDtypeStruct((1, _D), _F32),
                   jax.ShapeDtypeStruct((1, _D), _F32)),
    )(xsd, edge_feats, g3, wsd, we, wg, b1, k1c1, w2t, b2)

    mu2 = t1 / _E
    var2 = t2 / _E - mu2 * mu2
    k2 = p['e2_g'] * jax.lax.rsqrt(var2[0] + _EPS)
    c2 = p['e2_be'] - mu2[0] * k2
    k2c2 = jnp.concatenate([_row(k2), _row(c2)], axis=0)

    dst3 = dst.reshape(_NT, 1, _TE)
    he, gsum, gcnt = pl.pallas_call(
        _edge_final_body,
        grid=(_NT,),
        in_specs=[espec, pl.BlockSpec((1, 1, _TE), lambda i: (i, 0, 0)),
                  pl.BlockSpec((2, _D), lambda i: (0, 0))],
        out_specs=(espec,
                   pl.BlockSpec((1, _NPV, _D), lambda i: (i // _TPG, 0, 0)),
                   pl.BlockSpec((1, 1, _NPV), lambda i: (i // _TPG, 0, 0))),
        out_shape=(jax.ShapeDtypeStruct((_E, _D), _F32),
                   jax.ShapeDtypeStruct((_B, _NPV, _D), _F32),
                   jax.ShapeDtypeStruct((_B, 1, _NPV), _F32)),
    )(h2, dst3, k2c2)

    psums = gsum[:, :_NPG].reshape(_N, _D)
    cnt = gcnt[:, 0, :_NPG].reshape(_N)
    cntb = jnp.broadcast_to(cnt[:, None], (_N, _D))

    gp = jnp.zeros((16, _D), _F32).at[0:_B].set(global_feats)
    wn1t = p['n1_W'].T
    wg1t = p['g1_W'].T
    full = lambda shape: pl.BlockSpec(shape, lambda: tuple(0 for _ in shape))
    args = (node_feats, psums, cntb, gp,
            wn1t[0:128], wn1t[128:256], wn1t[256:384],
            _row(p['n1_b']), _row(p['n1_g']), _row(p['n1_be']),
            p['n2_W'].T, _row(p['n2_b']), _row(p['n2_g']), _row(p['n2_be']),
            wg1t[0:128], wg1t[128:256], wg1t[256:384],
            _row(p['g1_b']), _row(p['g1_g']), _row(p['g1_be']),
            p['g2_W'].T, _row(p['g2_b']), _row(p['g2_g']), _row(p['g2_be']))
    hn, hgp = pl.pallas_call(
        _node_global_body,
        in_specs=[full(a.shape) for a in args],
        out_specs=(full((_N, _D)), full((16, _D))),
        out_shape=(jax.ShapeDtypeStruct((_N, _D), _F32),
                   jax.ShapeDtypeStruct((16, _D), _F32)),
    )(*args)

    return hn, he, hgp[0:_B]


# speedup vs baseline: 1.1822x; 1.0504x over previous
import jax
import jax.numpy as jnp
from jax import lax
from jax.experimental import pallas as pl
from jax.experimental.pallas import tpu as pltpu
from jax.experimental.pallas import tpu_sc as plsc

_D = 128
_B = 10
_NPG = 1000
_EPG = 32000
_N = _B * _NPG
_E = _B * _EPG

_TE = 4000
_NT = _E // _TE
_TPG = _EPG // _TE

_NC = 2
_NS = 16
_NW = _NC * _NS
_CHUNK = 400
_PERW = _E // _NW
_NPV = 1024

_EPS = 1e-5
_BF = jnp.bfloat16
_F32 = jnp.float32


def _softplus(x):
    return jnp.maximum(x, 0.0) + jnp.log(1.0 + jnp.exp(-jnp.abs(x)))


def _bdot(a, b):
    return jnp.dot(a.astype(_BF), b.astype(_BF), preferred_element_type=_F32)


def _sc_gather(node_feats, src, dst):
    mesh = plsc.VectorSubcoreMesh(core_axis_name="c", subcore_axis_name="s")

    @pl.kernel(
        out_type=jax.ShapeDtypeStruct((_E, 2 * _D), _F32),
        mesh=mesh,
        scratch_types=[
            pltpu.VMEM((_CHUNK,), jnp.int32),
            pltpu.VMEM((_CHUNK, _D), _F32),
            pltpu.VMEM((_CHUNK,), jnp.int32),
            pltpu.VMEM((_CHUNK, _D), _F32),
        ],
    )
    def k(nf_hbm, src_hbm, dst_hbm, xsd_hbm, idx_s, rows_s, idx_d, rows_d):
        wid = lax.axis_index("s") * _NC + lax.axis_index("c")
        base = wid * _PERW

        @pl.loop(0, _PERW // _CHUNK)
        def _(j):
            off = base + j * _CHUNK
            pltpu.sync_copy(src_hbm.at[pl.ds(off, _CHUNK)], idx_s)
            pltpu.sync_copy(nf_hbm.at[idx_s], rows_s)
            pltpu.sync_copy(rows_s, xsd_hbm.at[pl.ds(off, _CHUNK), pl.ds(0, _D)])
            pltpu.sync_copy(dst_hbm.at[pl.ds(off, _CHUNK)], idx_d)
            pltpu.sync_copy(nf_hbm.at[idx_d], rows_d)
            pltpu.sync_copy(rows_d, xsd_hbm.at[pl.ds(off, _CHUNK), pl.ds(_D, _D)])

    return k(node_feats, src, dst)


def _edge_stats_body(xsd, ef, g3, wsd, we, wg, b1, s1, s2):
    i = pl.program_id(0)
    grow = _bdot(g3[0], wg[...]) + b1[...]
    h = _bdot(xsd[...], wsd[...]) + (_bdot(ef[...], we[...]) + grow)

    @pl.when(i == 0)
    def _():
        s1[...] = jnp.zeros_like(s1)
        s2[...] = jnp.zeros_like(s2)

    s1[...] += jnp.sum(h, axis=0, keepdims=True)
    s2[...] += jnp.sum(h * h, axis=0, keepdims=True)


def _edge_main_body(xsd, ef, g3, wsd, we, wg, b1, k1c1, w2, b2,
                    h2o, s1, s2):
    i = pl.program_id(0)
    grow = _bdot(g3[0], wg[...]) + b1[...]
    h = _bdot(xsd[...], wsd[...]) + (_bdot(ef[...], we[...]) + grow)
    a = _softplus(h * k1c1[0:1, :] + k1c1[1:2, :])
    h2 = _bdot(a, w2[...]) + b2[...]
    h2o[...] = h2

    @pl.when(i == 0)
    def _():
        s1[...] = jnp.zeros_like(s1)
        s2[...] = jnp.zeros_like(s2)

    s1[...] += jnp.sum(h2, axis=0, keepdims=True)
    s2[...] += jnp.sum(h2 * h2, axis=0, keepdims=True)


def _edge_final_body(h2, dstl, k2c2, he, psum, pcnt):
    i = pl.program_id(0)
    hev = _softplus(h2[...] * k2c2[0:1, :] + k2c2[1:2, :])
    he[...] = hev

    col = dstl[0, 0, :].reshape(_TE, 1) - (i // _TPG) * _NPG
    oh = (col == lax.broadcasted_iota(jnp.int32, (_TE, _NPV), 1))

    @pl.when(i % _TPG == 0)
    def _():
        psum[...] = jnp.zeros_like(psum)
        pcnt[...] = jnp.zeros_like(pcnt)

    dn = (((0,), (0,)), ((), ()))
    part = lax.dot_general(oh.astype(_BF), hev.astype(_BF), dn,
                           preferred_element_type=_F32)
    psum[0] += part
    pcnt[0] += jnp.sum(oh.astype(_F32), axis=0, keepdims=True)


def _node_global_body(nf, ssum, cntb, gp,
                      wn1a, wn1b, wn1c, bn1, gn1, ben1,
                      wn2, bn2, gn2, ben2,
                      wg1a, wg1b, wg1c, bg1, gg1, beg1,
                      wg2, bg2, gg2, beg2,
                      hn_o, hg_o):
    cnt = jnp.maximum(cntb[...], 1.0)
    have = ssum[...] / cnt

    r = lax.broadcasted_iota(jnp.int32, (_N, 16), 0)
    c = lax.broadcasted_iota(jnp.int32, (_N, 16), 1)
    oh = (r // _NPG == c).astype(_BF)

    gtab = _bdot(gp[...], wn1c[...])
    h1 = _bdot(nf[...], wn1a[...]) + _bdot(have, wn1b[...]) \
        + _bdot(oh, gtab) + bn1[...]
    mu = jnp.mean(h1, axis=0, keepdims=True)
    var = jnp.mean(h1 * h1, axis=0, keepdims=True) - mu * mu
    a1 = _softplus(gn1[...] * (h1 - mu) * jax.lax.rsqrt(var + _EPS) + ben1[...])

    h2 = _bdot(a1, wn2[...]) + bn2[...]
    mu2 = jnp.mean(h2, axis=0, keepdims=True)
    var2 = jnp.mean(h2 * h2, axis=0, keepdims=True) - mu2 * mu2
    hn = _softplus(gn2[...] * (h2 - mu2) * jax.lax.rsqrt(var2 + _EPS) + ben2[...])
    hn_o[...] = hn

    dn = (((0,), (0,)), ((), ()))
    han = lax.dot_general(oh, hn.astype(_BF), dn,
                          preferred_element_type=_F32) * (1.0 / _NPG)
    hae = lax.dot_general(oh, have.astype(_BF), dn,
                          preferred_element_type=_F32) * (1.0 / _NPG)

    hg1 = _bdot(han, wg1a[...]) + _bdot(hae, wg1b[...]) \
        + _bdot(gp[...], wg1c[...]) + bg1[...]
    m = (lax.broadcasted_iota(jnp.int32, (16, 384), 0) < _B).astype(_F32)
    mug = jnp.sum(hg1 * m, axis=0, keepdims=True) / _B
    varg = jnp.sum((hg1 - mug) * (hg1 - mug) * m, axis=0, keepdims=True) / _B
    ag = _softplus(gg1[...] * (hg1 - mug) * jax.lax.rsqrt(varg + _EPS) + beg1[...])

    hg2 = _bdot(ag, wg2[...]) + bg2[...]
    m2 = (lax.broadcasted_iota(jnp.int32, (16, _D), 0) < _B).astype(_F32)
    mug2 = jnp.sum(hg2 * m2, axis=0, keepdims=True) / _B
    varg2 = jnp.sum((hg2 - mug2) * (hg2 - mug2) * m2, axis=0, keepdims=True) / _B
    hg_o[...] = _softplus(gg2[...] * (hg2 - mug2) * jax.lax.rsqrt(varg2 + _EPS)
                          + beg2[...])


def _row(x):
    return x.reshape(1, -1)


def kernel(node_feats, edge_feats, global_feats, params, src, dst, node_gid,
           batch_num_nodes, batch_num_edges):
    p = params
    w1t = p['e1_W'].T
    ws, wd, we, wg = w1t[0:128], w1t[128:256], w1t[256:384], w1t[384:512]
    b1 = _row(p['e1_b'])
    w2t = p['e2_W'].T
    b2 = _row(p['e2_b'])

    g3 = global_feats.reshape(_B, 1, _D)

    xsd = _sc_gather(node_feats, src, dst)
    wsd = jnp.concatenate([ws, wd], axis=0)

    espec = pl.BlockSpec((_TE, _D), lambda i: (i, 0))
    espec2 = pl.BlockSpec((_TE, 2 * _D), lambda i: (i, 0))
    gspec = pl.BlockSpec((1, 1, _D), lambda i: (i // _TPG, 0, 0))
    wspec = pl.BlockSpec((_D, 512), lambda i: (0, 0))
    rspec = lambda n: pl.BlockSpec((1, n), lambda i: (0, 0))

    s1, s2 = pl.pallas_call(
        _edge_stats_body,
        grid=(_NT,),
        in_specs=[espec2, espec, gspec, pl.BlockSpec((2 * _D, 512), lambda i: (0, 0)),
                  wspec, wspec, rspec(512)],
        out_specs=(rspec(512), rspec(512)),
        out_shape=(jax.ShapeDtypeStruct((1, 512), _F32),
                   jax.ShapeDtypeStruct((1, 512), _F32)),
    )(xsd, edge_feats, g3, wsd, we, wg, b1)

    mu = s1 / _E
    var = s2 / _E - mu * mu
    k1 = p['e1_g'] * jax.lax.rsqrt(var[0] + _EPS)
    c1 = p['e1_be'] - mu[0] * k1
    k1c1 = jnp.concatenate([_row(k1), _row(c1)], axis=0)

    h2, t1, t2 = pl.pallas_call(
        _edge_main_body,
        grid=(_NT,),
        in_specs=[espec2, espec, gspec, pl.BlockSpec((2 * _D, 512), lambda i: (0, 0)),
                  wspec, wspec, rspec(512), pl.BlockSpec((2, 512), lambda i: (0, 0)),
                  pl.BlockSpec((512, _D), lambda i: (0, 0)), rspec(_D)],
        out_specs=(espec, rspec(_D), rspec(_D)),
        out_shape=(jax.ShapeDtypeStruct((_E, _D), _F32),
                   jax.ShapeDtypeStruct((1, _D), _F32),
                   jax.ShapeDtypeStruct((1, _D), _F32)),
    )(xsd, edge_feats, g3, wsd, we, wg, b1, k1c1, w2t, b2)

    mu2 = t1 / _E
    var2 = t2 / _E - mu2 * mu2
    k2 = p['e2_g'] * jax.lax.rsqrt(var2[0] + _EPS)
    c2 = p['e2_be'] - mu2[0] * k2
    k2c2 = jnp.concatenate([_row(k2), _row(c2)], axis=0)

    dst3 = dst.reshape(_NT, 1, _TE)
    he, gsum, gcnt = pl.pallas_call(
        _edge_final_body,
        grid=(_NT,),
        in_specs=[espec, pl.BlockSpec((1, 1, _TE), lambda i: (i, 0, 0)),
                  pl.BlockSpec((2, _D), lambda i: (0, 0))],
        out_specs=(espec,
                   pl.BlockSpec((1, _NPV, _D), lambda i: (i // _TPG, 0, 0)),
                   pl.BlockSpec((1, 1, _NPV), lambda i: (i // _TPG, 0, 0))),
        out_shape=(jax.ShapeDtypeStruct((_E, _D), _F32),
                   jax.ShapeDtypeStruct((_B, _NPV, _D), _F32),
                   jax.ShapeDtypeStruct((_B, 1, _NPV), _F32)),
    )(h2, dst3, k2c2)

    psums = gsum[:, :_NPG].reshape(_N, _D)
    cnt = gcnt[:, 0, :_NPG].reshape(_N)
    cntb = jnp.broadcast_to(cnt[:, None], (_N, _D))

    gp = jnp.zeros((16, _D), _F32).at[0:_B].set(global_feats)
    wn1t = p['n1_W'].T
    wg1t = p['g1_W'].T
    full = lambda shape: pl.BlockSpec(shape, lambda: tuple(0 for _ in shape))
    args = (node_feats, psums, cntb, gp,
            wn1t[0:128], wn1t[128:256], wn1t[256:384],
            _row(p['n1_b']), _row(p['n1_g']), _row(p['n1_be']),
            p['n2_W'].T, _row(p['n2_b']), _row(p['n2_g']), _row(p['n2_be']),
            wg1t[0:128], wg1t[128:256], wg1t[256:384],
            _row(p['g1_b']), _row(p['g1_g']), _row(p['g1_be']),
            p['g2_W'].T, _row(p['g2_b']), _row(p['g2_g']), _row(p['g2_be']))
    hn, hgp = pl.pallas_call(
        _node_global_body,
        in_specs=[full(a.shape) for a in args],
        out_specs=(full((_N, _D)), full((16, _D))),
        out_shape=(jax.ShapeDtypeStruct((_N, _D), _F32),
                   jax.ShapeDtypeStruct((16, _D), _F32)),
    )(*args)

    return hn, he, hgp[0:_B]


# speedup vs baseline: 1.2201x; 1.0321x over previous
import jax
import jax.numpy as jnp
from jax import lax
from jax.experimental import pallas as pl
from jax.experimental.pallas import tpu as pltpu
from jax.experimental.pallas import tpu_sc as plsc

_D = 128
_B = 10
_NPG = 1000
_EPG = 32000
_N = _B * _NPG
_E = _B * _EPG

_TE = 4000
_NT = _E // _TE
_TPG = _EPG // _TE

_NC = 2
_NS = 16
_NW = _NC * _NS
_CHUNK = 400
_PERW = _E // _NW
_NPV = 1024

_EPS = 1e-5
_BF = jnp.bfloat16
_F32 = jnp.float32


def _softplus(x):
    return jnp.maximum(x, 0.0) + jnp.log(1.0 + jnp.exp(-jnp.abs(x)))


def _bdot(a, b):
    return jnp.dot(a.astype(_BF), b.astype(_BF), preferred_element_type=_F32)


def _sc_gather(node_feats, src, dst):
    mesh = plsc.VectorSubcoreMesh(core_axis_name="c", subcore_axis_name="s")

    @pl.kernel(
        out_type=jax.ShapeDtypeStruct((_E, 2 * _D), _F32),
        mesh=mesh,
        scratch_types=[
            pltpu.VMEM((_CHUNK,), jnp.int32),
            pltpu.VMEM((_CHUNK, _D), _F32),
            pltpu.VMEM((_CHUNK,), jnp.int32),
            pltpu.VMEM((_CHUNK, _D), _F32),
            pltpu.SemaphoreType.DMA,
            pltpu.SemaphoreType.DMA,
            pltpu.SemaphoreType.DMA,
            pltpu.SemaphoreType.DMA,
        ],
    )
    def k(nf_hbm, src_hbm, dst_hbm, xsd_hbm, idx_s, rows_s, idx_d, rows_d,
          sa, sb, wa, wb):
        wid = lax.axis_index("s") * _NC + lax.axis_index("c")
        base = wid * _PERW

        @pl.loop(0, _PERW // _CHUNK)
        def _(j):
            off = base + j * _CHUNK
            la = pltpu.async_copy(src_hbm.at[pl.ds(off, _CHUNK)], idx_s, sa)
            lb = pltpu.async_copy(dst_hbm.at[pl.ds(off, _CHUNK)], idx_d, sb)
            la.wait()
            lb.wait()

            @pl.when(j > 0)
            def _():
                pltpu.make_async_copy(
                    rows_s, xsd_hbm.at[pl.ds(off, _CHUNK), pl.ds(0, _D)], wa).wait()
                pltpu.make_async_copy(
                    rows_d, xsd_hbm.at[pl.ds(off, _CHUNK), pl.ds(_D, _D)], wb).wait()

            ga = pltpu.async_copy(nf_hbm.at[idx_s], rows_s, sa)
            gb = pltpu.async_copy(nf_hbm.at[idx_d], rows_d, sb)
            ga.wait()
            gb.wait()
            pltpu.async_copy(rows_s, xsd_hbm.at[pl.ds(off, _CHUNK), pl.ds(0, _D)], wa)
            pltpu.async_copy(rows_d, xsd_hbm.at[pl.ds(off, _CHUNK), pl.ds(_D, _D)], wb)

        pltpu.make_async_copy(
            rows_s, xsd_hbm.at[pl.ds(base, _CHUNK), pl.ds(0, _D)], wa).wait()
        pltpu.make_async_copy(
            rows_d, xsd_hbm.at[pl.ds(base, _CHUNK), pl.ds(_D, _D)], wb).wait()

    return k(node_feats, src, dst)


def _edge_stats_body(xsd, ef, g3, wsd, we, wg, b1, s1, s2):
    i = pl.program_id(0)
    grow = _bdot(g3[0], wg[...]) + b1[...]
    h = _bdot(xsd[...], wsd[...]) + (_bdot(ef[...], we[...]) + grow)

    @pl.when(i == 0)
    def _():
        s1[...] = jnp.zeros_like(s1)
        s2[...] = jnp.zeros_like(s2)

    s1[...] += jnp.sum(h, axis=0, keepdims=True)
    s2[...] += jnp.sum(h * h, axis=0, keepdims=True)


def _edge_main_body(xsd, ef, g3, wsd, we, wg, b1, k1c1, w2, b2,
                    h2o, s1, s2):
    i = pl.program_id(0)
    grow = _bdot(g3[0], wg[...]) + b1[...]
    h = _bdot(xsd[...], wsd[...]) + (_bdot(ef[...], we[...]) + grow)
    a = _softplus(h * k1c1[0:1, :] + k1c1[1:2, :])
    h2 = _bdot(a, w2[...]) + b2[...]
    h2o[...] = h2

    @pl.when(i == 0)
    def _():
        s1[...] = jnp.zeros_like(s1)
        s2[...] = jnp.zeros_like(s2)

    s1[...] += jnp.sum(h2, axis=0, keepdims=True)
    s2[...] += jnp.sum(h2 * h2, axis=0, keepdims=True)


def _edge_final_body(h2, dstl, k2c2, he, psum, pcnt):
    i = pl.program_id(0)
    hev = _softplus(h2[...] * k2c2[0:1, :] + k2c2[1:2, :])
    he[...] = hev

    col = dstl[0, 0, :].reshape(_TE, 1) - (i // _TPG) * _NPG
    oh = (col == lax.broadcasted_iota(jnp.int32, (_TE, _NPV), 1))

    @pl.when(i % _TPG == 0)
    def _():
        psum[...] = jnp.zeros_like(psum)
        pcnt[...] = jnp.zeros_like(pcnt)

    dn = (((0,), (0,)), ((), ()))
    part = lax.dot_general(oh.astype(_BF), hev.astype(_BF), dn,
                           preferred_element_type=_F32)
    psum[0] += part
    pcnt[0] += jnp.sum(oh.astype(_F32), axis=0, keepdims=True)


def _node_global_body(nf, ssum, cntb, gp,
                      wn1a, wn1b, wn1c, bn1, gn1, ben1,
                      wn2, bn2, gn2, ben2,
                      wg1a, wg1b, wg1c, bg1, gg1, beg1,
                      wg2, bg2, gg2, beg2,
                      hn_o, hg_o):
    cnt = jnp.maximum(cntb[...], 1.0)
    have = ssum[...] / cnt

    r = lax.broadcasted_iota(jnp.int32, (_N, 16), 0)
    c = lax.broadcasted_iota(jnp.int32, (_N, 16), 1)
    oh = (r // _NPG == c).astype(_BF)

    gtab = _bdot(gp[...], wn1c[...])
    h1 = _bdot(nf[...], wn1a[...]) + _bdot(have, wn1b[...]) \
        + _bdot(oh, gtab) + bn1[...]
    mu = jnp.mean(h1, axis=0, keepdims=True)
    var = jnp.mean(h1 * h1, axis=0, keepdims=True) - mu * mu
    a1 = _softplus(gn1[...] * (h1 - mu) * jax.lax.rsqrt(var + _EPS) + ben1[...])

    h2 = _bdot(a1, wn2[...]) + bn2[...]
    mu2 = jnp.mean(h2, axis=0, keepdims=True)
    var2 = jnp.mean(h2 * h2, axis=0, keepdims=True) - mu2 * mu2
    hn = _softplus(gn2[...] * (h2 - mu2) * jax.lax.rsqrt(var2 + _EPS) + ben2[...])
    hn_o[...] = hn

    dn = (((0,), (0,)), ((), ()))
    han = lax.dot_general(oh, hn.astype(_BF), dn,
                          preferred_element_type=_F32) * (1.0 / _NPG)
    hae = lax.dot_general(oh, have.astype(_BF), dn,
                          preferred_element_type=_F32) * (1.0 / _NPG)

    hg1 = _bdot(han, wg1a[...]) + _bdot(hae, wg1b[...]) \
        + _bdot(gp[...], wg1c[...]) + bg1[...]
    m = (lax.broadcasted_iota(jnp.int32, (16, 384), 0) < _B).astype(_F32)
    mug = jnp.sum(hg1 * m, axis=0, keepdims=True) / _B
    varg = jnp.sum((hg1 - mug) * (hg1 - mug) * m, axis=0, keepdims=True) / _B
    ag = _softplus(gg1[...] * (hg1 - mug) * jax.lax.rsqrt(varg + _EPS) + beg1[...])

    hg2 = _bdot(ag, wg2[...]) + bg2[...]
    m2 = (lax.broadcasted_iota(jnp.int32, (16, _D), 0) < _B).astype(_F32)
    mug2 = jnp.sum(hg2 * m2, axis=0, keepdims=True) / _B
    varg2 = jnp.sum((hg2 - mug2) * (hg2 - mug2) * m2, axis=0, keepdims=True) / _B
    hg_o[...] = _softplus(gg2[...] * (hg2 - mug2) * jax.lax.rsqrt(varg2 + _EPS)
                          + beg2[...])


def _row(x):
    return x.reshape(1, -1)


def kernel(node_feats, edge_feats, global_feats, params, src, dst, node_gid,
           batch_num_nodes, batch_num_edges):
    p = params
    w1t = p['e1_W'].T
    ws, wd, we, wg = w1t[0:128], w1t[128:256], w1t[256:384], w1t[384:512]
    b1 = _row(p['e1_b'])
    w2t = p['e2_W'].T
    b2 = _row(p['e2_b'])

    g3 = global_feats.reshape(_B, 1, _D)

    xsd = _sc_gather(node_feats, src, dst)
    wsd = jnp.concatenate([ws, wd], axis=0)

    espec = pl.BlockSpec((_TE, _D), lambda i: (i, 0))
    espec2 = pl.BlockSpec((_TE, 2 * _D), lambda i: (i, 0))
    gspec = pl.BlockSpec((1, 1, _D), lambda i: (i // _TPG, 0, 0))
    wspec = pl.BlockSpec((_D, 512), lambda i: (0, 0))
    rspec = lambda n: pl.BlockSpec((1, n), lambda i: (0, 0))

    s1, s2 = pl.pallas_call(
        _edge_stats_body,
        grid=(_NT,),
        in_specs=[espec2, espec, gspec, pl.BlockSpec((2 * _D, 512), lambda i: (0, 0)),
                  wspec, wspec, rspec(512)],
        out_specs=(rspec(512), rspec(512)),
        out_shape=(jax.ShapeDtypeStruct((1, 512), _F32),
                   jax.ShapeDtypeStruct((1, 512), _F32)),
    )(xsd, edge_feats, g3, wsd, we, wg, b1)

    mu = s1 / _E
    var = s2 / _E - mu * mu
    k1 = p['e1_g'] * jax.lax.rsqrt(var[0] + _EPS)
    c1 = p['e1_be'] - mu[0] * k1
    k1c1 = jnp.concatenate([_row(k1), _row(c1)], axis=0)

    h2, t1, t2 = pl.pallas_call(
        _edge_main_body,
        grid=(_NT,),
        in_specs=[espec2, espec, gspec, pl.BlockSpec((2 * _D, 512), lambda i: (0, 0)),
                  wspec, wspec, rspec(512), pl.BlockSpec((2, 512), lambda i: (0, 0)),
                  pl.BlockSpec((512, _D), lambda i: (0, 0)), rspec(_D)],
        out_specs=(espec, rspec(_D), rspec(_D)),
        out_shape=(jax.ShapeDtypeStruct((_E, _D), _F32),
                   jax.ShapeDtypeStruct((1, _D), _F32),
                   jax.ShapeDtypeStruct((1, _D), _F32)),
    )(xsd, edge_feats, g3, wsd, we, wg, b1, k1c1, w2t, b2)

    mu2 = t1 / _E
    var2 = t2 / _E - mu2 * mu2
    k2 = p['e2_g'] * jax.lax.rsqrt(var2[0] + _EPS)
    c2 = p['e2_be'] - mu2[0] * k2
    k2c2 = jnp.concatenate([_row(k2), _row(c2)], axis=0)

    dst3 = dst.reshape(_NT, 1, _TE)
    he, gsum, gcnt = pl.pallas_call(
        _edge_final_body,
        grid=(_NT,),
        in_specs=[espec, pl.BlockSpec((1, 1, _TE), lambda i: (i, 0, 0)),
                  pl.BlockSpec((2, _D), lambda i: (0, 0))],
        out_specs=(espec,
                   pl.BlockSpec((1, _NPV, _D), lambda i: (i // _TPG, 0, 0)),
                   pl.BlockSpec((1, 1, _NPV), lambda i: (i // _TPG, 0, 0))),
        out_shape=(jax.ShapeDtypeStruct((_E, _D), _F32),
                   jax.ShapeDtypeStruct((_B, _NPV, _D), _F32),
                   jax.ShapeDtypeStruct((_B, 1, _NPV), _F32)),
    )(h2, dst3, k2c2)

    psums = gsum[:, :_NPG].reshape(_N, _D)
    cnt = gcnt[:, 0, :_NPG].reshape(_N)
    cntb = jnp.broadcast_to(cnt[:, None], (_N, _D))

    gp = jnp.zeros((16, _D), _F32).at[0:_B].set(global_feats)
    wn1t = p['n1_W'].T
    wg1t = p['g1_W'].T
    full = lambda shape: pl.BlockSpec(shape, lambda: tuple(0 for _ in shape))
    args = (node_feats, psums, cntb, gp,
            wn1t[0:128], wn1t[128:256], wn1t[256:384],
            _row(p['n1_b']), _row(p['n1_g']), _row(p['n1_be']),
            p['n2_W'].T, _row(p['n2_b']), _row(p['n2_g']), _row(p['n2_be']),
            wg1t[0:128], wg1t[128:256], wg1t[256:384],
            _row(p['g1_b']), _row(p['g1_g']), _row(p['g1_be']),
            p['g2_W'].T, _row(p['g2_b']), _row(p['g2_g']), _row(p['g2_be']))
    hn, hgp = pl.pallas_call(
        _node_global_body,
        in_specs=[full(a.shape) for a in args],
        out_specs=(full((_N, _D)), full((16, _D))),
        out_shape=(jax.ShapeDtypeStruct((_N, _D), _F32),
                   jax.ShapeDtypeStruct((16, _D), _F32)),
    )(*args)

    return hn, he, hgp[0:_B]
